# baseline (speedup 1.0000x reference)
import functools

import jax
import jax.numpy as jnp
from jax import lax
from jax.experimental import pallas as pl
from jax.experimental.pallas import tpu as pltpu
from jax.experimental.pallas import tpu_sc as plsc

B, F, V, D = 16384, 26, 1000000, 16
OUT_W = F * D + 1
L = 16
RPS = 128 // D

NC, NS = 2, 16
NW = NC * NS
ROWS_PER_TILE = B // NW
CB = 16
NG = CB * F
NCHUNK = ROWS_PER_TILE // CB

NVT = (V + 127) // 128
NVT_FULL = V // 128
VTAIL = V - NVT_FULL * 128
TPW = (NVT + NW - 1) // NW


def _xpose_body(tok_t_hbm, tail_hbm, out_hbm, tin_v, tout_v):
    wid = lax.axis_index("s") * NC + lax.axis_index("c")
    t0 = wid * TPW
    t1 = jnp.minimum(t0 + TPW, NVT)
    riota = lax.iota(jnp.int32, L)
    rowhalf = riota // 8
    colbase = (riota % 8) * D

    def tile_loop(t, carry):
        @pl.when(t < NVT_FULL)
        def _full():
            pltpu.sync_copy(tok_t_hbm.at[:, pl.ds(t * 128, 128)], tin_v)
            for d in range(D):
                for g in range(8):
                    v16 = tin_v[d, pl.ds(g * L, L)]
                    plsc.store_scatter(
                        tout_v, [2 * g + rowhalf, colbase + d], v16)
            pltpu.sync_copy(tout_v, out_hbm.at[pl.ds(t * L, L), :])

        @pl.when(t == NVT_FULL)
        def _tail():
            pltpu.sync_copy(tail_hbm, tin_v)
            for d in range(D):
                for g in range(VTAIL // L):
                    v16 = tin_v[d, pl.ds(g * L, L)]
                    plsc.store_scatter(
                        tout_v, [2 * g + rowhalf, colbase + d], v16)
            pltpu.sync_copy(tout_v.at[pl.ds(0, VTAIL * D // 128), :],
                            out_hbm.at[pl.ds(NVT_FULL * L, VTAIL * D // 128), :])

        return carry

    lax.fori_loop(t0, t1, tile_loop, 0, unroll=False)


def _tile_body(idx_hbm, tok_hbm, fo_hbm, bias_hbm, pat_hbm, out_hbm,
               idx_v, hi_v, wide_v, fo_v, out_v, bias_v, pat_v,
               sem_tok, sem_fo):
    wid = lax.axis_index("s") * NC + lax.axis_index("c")
    tile_base = wid * ROWS_PER_TILE
    pltpu.sync_copy(bias_hbm, bias_v)
    pltpu.sync_copy(pat_hbm, pat_v)
    bias_vec = bias_v[...]
    riota = lax.iota(jnp.int32, L)

    def chunk(c, carry):
        base = tile_base + c * CB
        pltpu.sync_copy(idx_hbm.at[pl.ds(base * F, NG)], idx_v)

        def shr(k, carry2):
            v = idx_v[pl.ds(k * L, L)]
            hi_v[pl.ds(k * L, L)] = lax.shift_right_logical(v, 3)
            return carry2

        lax.fori_loop(0, NG // L, shr, 0, unroll=True)
        cp_tok = pltpu.async_copy(tok_hbm.at[hi_v], wide_v, sem_tok)
        cp_fo = pltpu.async_copy(fo_hbm.at[idx_v], fo_v, sem_fo)
        cp_tok.wait()
        cp_fo.wait()

        for g in range(NG // L):
            jvec = g * L + riota
            idxvec = idx_v[pl.ds(g * L, L)]
            offvec = (idxvec & (RPS - 1)) * D
            dstvec = pat_v[pl.ds(g * L, L)]
            for d in range(D):
                v = plsc.load_gather(wide_v, [jvec, offvec + d])
                plsc.store_scatter(out_v, [dstvec + d], v)

        acc = bias_vec
        for f in range(F):
            acc = acc + plsc.load_gather(fo_v, [riota * F + f])
        plsc.store_scatter(out_v, [riota * OUT_W + F * D], acc)

        pltpu.sync_copy(out_v, out_hbm.at[pl.ds(base * OUT_W, CB * OUT_W)])
        return carry

    lax.fori_loop(0, NCHUNK, chunk, 0, unroll=False)


@jax.jit
def _run(idx_flat, tok_t, tok_tail, fo_flat, bias16, pat):
    mesh = plsc.VectorSubcoreMesh(core_axis_name="c", subcore_axis_name="s",
                                  num_cores=NC, num_subcores=NS)
    xpose = functools.partial(
        pl.kernel,
        mesh=mesh,
        out_type=jax.ShapeDtypeStruct((V // RPS, RPS * D), jnp.float32),
        scratch_types=[
            pltpu.VMEM((D, 128), jnp.float32),
            pltpu.VMEM((L, 128), jnp.float32),
        ],
        compiler_params=pltpu.CompilerParams(
            needs_layout_passes=False, use_tc_tiling_on_sc=True),
    )(_xpose_body)
    tok_wide = xpose(tok_t, tok_tail)

    k = functools.partial(
        pl.kernel,
        mesh=mesh,
        out_type=jax.ShapeDtypeStruct((B * OUT_W,), jnp.float32),
        scratch_types=[
            pltpu.VMEM((NG,), jnp.int32),
            pltpu.VMEM((NG,), jnp.int32),
            pltpu.VMEM((NG, RPS * D), jnp.float32),
            pltpu.VMEM((NG,), jnp.float32),
            pltpu.VMEM((CB * OUT_W,), jnp.float32),
            pltpu.VMEM((L,), jnp.float32),
            pltpu.VMEM((NG,), jnp.int32),
            pltpu.SemaphoreType.DMA,
            pltpu.SemaphoreType.DMA,
        ],
        compiler_params=pltpu.CompilerParams(
            needs_layout_passes=False, use_tc_tiling_on_sc=True),
    )(_tile_body)
    return k(idx_flat, tok_wide, fo_flat, bias16, pat)


def kernel(indices, token_table, first_order_table, first_order_bias):
    idx_flat = indices.reshape(-1)
    tok_t = token_table.T
    tok_tail = jnp.pad(token_table[V - VTAIL:].T, ((0, 0), (0, 128 - VTAIL)))
    fo_flat = first_order_table.reshape(-1)
    bias16 = jnp.broadcast_to(first_order_bias, (L,))
    j = jnp.arange(NG, dtype=jnp.int32)
    pat = (j // F) * OUT_W + (j % F) * D
    return _run(idx_flat, tok_t, tok_tail, fo_flat, bias16, pat).reshape(B, OUT_W)


# speedup vs baseline: 1.9235x; 1.9235x over previous
import functools

import jax
import jax.numpy as jnp
from jax import lax
from jax.experimental import pallas as pl
from jax.experimental.pallas import tpu as pltpu
from jax.experimental.pallas import tpu_sc as plsc

B, F, V, D = 16384, 26, 1000000, 16
OUT_W = F * D + 1
L = 16
RPS = 128 // D

NC, NS = 2, 16
NW = NC * NS
ROWS_PER_TILE = B // NW
CB = 64
NCHUNK = ROWS_PER_TILE // CB

NVT = (V + 127) // 128
NVT_FULL = V // 128
VTAIL = V - NVT_FULL * 128
TPW = NVT_FULL // NW - (NVT_FULL // NW) % 2
NREST = NVT - TPW * NW


def _xpose_tile(tin, tout, rowhalf, colbase, ngroups):
    for d in range(D):
        for g in range(ngroups):
            v16 = tin[d, pl.ds(g * L, L)]
            plsc.store_scatter(tout, [2 * g + rowhalf, colbase + d], v16)


def _xpose_body(tok_t_hbm, tail_hbm, out_hbm,
                tin0, tin1, tout0, tout1,
                sem_i0, sem_i1, sem_o0, sem_o1):
    wid = lax.axis_index("s") * NC + lax.axis_index("c")
    tbase = wid * TPW
    riota = lax.iota(jnp.int32, L)
    rowhalf = riota // 8
    colbase = (riota % 8) * D
    tins = (tin0, tin1)
    touts = (tout0, tout1)
    sem_is = (sem_i0, sem_i1)
    sem_os = (sem_o0, sem_o1)

    def in_src(t):
        return tok_t_hbm.at[:, pl.ds(t * 128, 128)]

    def out_dst(t):
        return out_hbm.at[pl.ds(t * L, L), :]

    pltpu.async_copy(in_src(tbase), tin0, sem_i0)
    pltpu.async_copy(in_src(tbase + 1), tin1, sem_i1)

    def step(i, carry):
        for bidx in range(2):
            s = 2 * i + bidx
            t = tbase + s
            tin, tout = tins[bidx], touts[bidx]
            sem_i, sem_o = sem_is[bidx], sem_os[bidx]
            pltpu.make_async_copy(in_src(t), tin, sem_i).wait()

            @pl.when(s >= 2)
            def _drain():
                pltpu.make_async_copy(tout, out_dst(t), sem_o).wait()

            _xpose_tile(tin, tout, rowhalf, colbase, 8)
            pltpu.async_copy(tout, out_dst(t), sem_o)

            @pl.when(s + 2 < TPW)
            def _next():
                pltpu.async_copy(in_src(t + 2), tin, sem_i)

        return carry

    lax.fori_loop(0, TPW // 2, step, 0, unroll=False)
    pltpu.make_async_copy(tout0, out_dst(tbase + TPW - 2), sem_o0).wait()
    pltpu.make_async_copy(tout1, out_dst(tbase + TPW - 1), sem_o1).wait()

    @pl.when(wid < NREST - 1)
    def _rest_full():
        t = NW * TPW + wid
        pltpu.sync_copy(in_src(t), tin0)
        _xpose_tile(tin0, tout0, rowhalf, colbase, 8)
        pltpu.sync_copy(tout0, out_dst(t))

    @pl.when(wid == NREST - 1)
    def _rest_tail():
        pltpu.sync_copy(tail_hbm, tin0)
        _xpose_tile(tin0, tout0, rowhalf, colbase, VTAIL // L)
        pltpu.sync_copy(tout0.at[pl.ds(0, VTAIL * D // 128), :],
                        out_hbm.at[pl.ds(NVT_FULL * L, VTAIL * D // 128), :])


def _gather_body(idx_hbm, tok_hbm, fo_hbm, bias_hbm, out_hbm,
                 idx_v, rows_v, fo_v, out_v, bias_v, sem_tok, sem_fo):
    wid = lax.axis_index("s") * NC + lax.axis_index("c")
    tile_base = wid * ROWS_PER_TILE
    pltpu.sync_copy(bias_hbm, bias_v)
    bias_vec = bias_v[...]
    riota = lax.iota(jnp.int32, L)

    def chunk(c, carry):
        base = tile_base + c * CB
        pltpu.sync_copy(idx_hbm.at[pl.ds(base * F, CB * F)], idx_v)
        cp_tok = pltpu.async_copy(tok_hbm.at[idx_v], rows_v, sem_tok)
        cp_fo = pltpu.async_copy(fo_hbm.at[idx_v], fo_v, sem_fo)
        cp_tok.wait()
        cp_fo.wait()

        def row(b, carry2):
            for f in range(F):
                out_v[b, pl.ds(f * D, D)] = rows_v[b * F + f]
            return carry2

        lax.fori_loop(0, CB, row, 0, unroll=False)

        def grp(g, carry2):
            b0 = g * L
            acc = bias_vec
            for f in range(F):
                acc = acc + plsc.load_gather(fo_v, [(b0 + riota) * F + f])
            plsc.store_scatter(
                out_v, [b0 + riota, jnp.full((L,), F * D, jnp.int32)], acc)
            return carry2

        lax.fori_loop(0, CB // L, grp, 0, unroll=False)

        pltpu.sync_copy(out_v, out_hbm.at[pl.ds(base, CB)])
        return carry

    lax.fori_loop(0, NCHUNK, chunk, 0, unroll=False)


@jax.jit
def _run(idx_flat, tok_t, tok_tail, fo_flat, bias16):
    mesh = plsc.VectorSubcoreMesh(core_axis_name="c", subcore_axis_name="s",
                                  num_cores=NC, num_subcores=NS)
    xpose = functools.partial(
        pl.kernel,
        mesh=mesh,
        out_type=jax.ShapeDtypeStruct((V // RPS, RPS * D), jnp.float32),
        scratch_types=[
            pltpu.VMEM((D, 128), jnp.float32),
            pltpu.VMEM((D, 128), jnp.float32),
            pltpu.VMEM((L, 128), jnp.float32),
            pltpu.VMEM((L, 128), jnp.float32),
            pltpu.SemaphoreType.DMA,
            pltpu.SemaphoreType.DMA,
            pltpu.SemaphoreType.DMA,
            pltpu.SemaphoreType.DMA,
        ],
        compiler_params=pltpu.CompilerParams(
            needs_layout_passes=False, use_tc_tiling_on_sc=True),
    )(_xpose_body)
    tok_rm = xpose(tok_t, tok_tail).reshape(V, D)

    k = functools.partial(
        pl.kernel,
        mesh=mesh,
        out_type=jax.ShapeDtypeStruct((B, OUT_W), jnp.float32),
        scratch_types=[
            pltpu.VMEM((CB * F,), jnp.int32),
            pltpu.VMEM((CB * F, D), jnp.float32),
            pltpu.VMEM((CB * F,), jnp.float32),
            pltpu.VMEM((CB, OUT_W), jnp.float32),
            pltpu.VMEM((L,), jnp.float32),
            pltpu.SemaphoreType.DMA,
            pltpu.SemaphoreType.DMA,
        ],
        compiler_params=pltpu.CompilerParams(
            needs_layout_passes=False, use_tc_tiling_on_sc=False),
    )(_gather_body)
    return k(idx_flat, tok_rm, fo_flat, bias16)


def kernel(indices, token_table, first_order_table, first_order_bias):
    idx_flat = indices.reshape(-1)
    tok_t = token_table.T
    tok_tail = jnp.pad(token_table[V - VTAIL:].T, ((0, 0), (0, 128 - VTAIL)))
    fo_flat = first_order_table.reshape(-1)
    bias16 = jnp.broadcast_to(first_order_bias, (L,))
    return _run(idx_flat, tok_t, tok_tail, fo_flat, bias16)
